# baseline (speedup 1.0000x reference)
import jax
import jax.numpy as jnp
import numpy as np
from jax.experimental import pallas as pl

N = 10000
D = 128
BLK = 1000


def _dense_body(h_ref, t1_ref, t2_ref, w0_ref, w1_ref, w2_ref, b_ref, out_ref):
    out_ref[...] = (
        jnp.dot(h_ref[...], w0_ref[...], preferred_element_type=jnp.float32)
        + jnp.dot(t1_ref[...], w1_ref[...], preferred_element_type=jnp.float32)
        + jnp.dot(t2_ref[...], w2_ref[...], preferred_element_type=jnp.float32)
        + b_ref[...])


def _dense_stage(h, t1, t2, W, b):
    grid = (N // BLK,)
    bspec = pl.BlockSpec((BLK, D), lambda i: (i, 0))
    wspec = pl.BlockSpec((D, D), lambda i: (0, 0))
    out = pl.pallas_call(
        _dense_body,
        grid=grid,
        in_specs=[bspec, bspec, bspec, wspec, wspec, wspec,
                  pl.BlockSpec((1, D), lambda i: (0, 0))],
        out_specs=bspec,
        out_shape=jax.ShapeDtypeStruct((N, D), jnp.float32),
    )(h, t1, t2, W[0], W[1], W[2], b.reshape(1, D))
    return out


def kernel(h, edges, W, b, p):
    row, col = edges[0], edges[1]
    ones = jnp.ones((row.shape[0],), dtype=jnp.float32)
    deg = jax.ops.segment_sum(ones, row, num_segments=N)
    dinv = jnp.where(deg > 0, jax.lax.rsqrt(jnp.maximum(deg, 1e-12)), 0.0)
    w = -dinv[row] * dinv[col]

    def lhat(x):
        msg = w[:, None] * jnp.take(x, row, axis=0)
        return jax.ops.segment_sum(msg, col, num_segments=N)

    Tx1 = lhat(h)
    Tx2 = 2.0 * lhat(Tx1) - h

    out = _dense_stage(h, Tx1, Tx2, W, b)
    hh = jax.nn.elu(out)
    score = hh @ p / jnp.linalg.norm(p)

    kk = int(np.ceil(0.5 * N))
    vals, idx = jax.lax.top_k(score, kk)
    h_out = jnp.take(hh, idx, axis=0) * jnp.tanh(vals)[:, None]
    return (h_out, idx)
